# baseline (speedup 1.0000x reference)
import jax
import jax.numpy as jnp
import numpy as np
from jax import lax
from jax.experimental import pallas as pl
from jax.experimental.pallas import tpu as pltpu
from jax.experimental.pallas import tpu_sc as plsc

N = 10000
C = 128
K = 32
HIDDEN = 128
ST = 16
BN = 4
NBLK = N // BN
NW = 32
TL = 80
NB = 8
LANES = 16
CCH = C // 32

_PERM = np.empty(C, np.int32)
for _c in range(CCH):
  for _q in range(32):
    _PERM[32 * _c + _q] = 32 * _c + (2 * _q if _q < 16 else 2 * (_q - 16) + 1)


def _sc_gather_body(knn_hbm, feat_hbm, stat_hbm, aggp_hbm, ssum_hbm,
                    idx_all, *bufs):
  cid = lax.axis_index("c")
  sid = lax.axis_index("s")
  wid = sid * 2 + cid
  base = jnp.minimum(wid * TL, NBLK - TL)

  rows_b = bufs[0:NB]
  srows_b = bufs[NB:2 * NB]
  agg_b = bufs[2 * NB:3 * NB]
  st_b = bufs[3 * NB:4 * NB]
  semg = bufs[4 * NB:5 * NB]
  semo = bufs[5 * NB:6 * NB]

  pltpu.sync_copy(knn_hbm.at[pl.ds(base, TL)], idx_all)

  def gathers(t, p):
    return (
        pltpu.make_async_copy(feat_hbm.at[idx_all.at[t]], rows_b[p], semg[p]),
        pltpu.make_async_copy(stat_hbm.at[idx_all.at[t]], srows_b[p], semg[p]),
    )

  def out_copies(t, p):
    return (
        pltpu.make_async_copy(
            agg_b[p], aggp_hbm.at[pl.ds((base + t) * BN, BN)], semo[p]),
        pltpu.make_async_copy(
            st_b[p], ssum_hbm.at[pl.ds((base + t) * BN, BN)], semo[p]),
    )

  for tp in range(NB - 1):
    for cp in gathers(tp, tp):
      cp.start()

  def outer(g, _):
    for phase in range(NB):
      t = g * NB + phase
      tn = t + (NB - 1)

      @pl.when(tn < TL)
      def _():
        for cp in gathers(tn, (phase + NB - 1) % NB):
          cp.start()

      @pl.when(t >= NB)
      def _():
        for cp in out_copies(t - NB, phase):
          cp.wait()

      for cp in gathers(t, phase):
        cp.wait()

      rows_v = rows_b[phase]
      srows_v = srows_b[phase]
      for n in range(BN):
        def red(j, carry):
          r = n * K + j
          new = []
          for c in range(CCH):
            w = rows_v[r, pl.ds(16 * c, 16)]
            e, o = plsc.unpack(plsc.bitcast(w, jnp.bfloat16),
                               format=plsc.PackFormat.INTERLEAVED)
            new.append(carry[2 * c] + e)
            new.append(carry[2 * c + 1] + o)
          new.append(carry[2 * CCH] + srows_v[r, :])
          return tuple(new)

        zero = jnp.zeros((LANES,), jnp.float32)
        init = tuple(zero for _ in range(2 * CCH + 1))
        out = lax.fori_loop(0, K, red, init, unroll=2)
        for c in range(CCH):
          agg_b[phase][n, pl.ds(32 * c, LANES)] = out[2 * c]
          agg_b[phase][n, pl.ds(32 * c + LANES, LANES)] = out[2 * c + 1]
        st_b[phase][n, :] = out[2 * CCH]

      for cp in out_copies(t, phase):
        cp.start()

    return _

  lax.fori_loop(0, TL // NB, outer, None)

  for p in range(NB):
    for cp in out_copies(0, p):
      cp.wait()


def _sc_gather(knn2d, feati, stat16):
  mesh = plsc.VectorSubcoreMesh(core_axis_name="c", subcore_axis_name="s")
  return pl.kernel(
      _sc_gather_body,
      out_type=(
          jax.ShapeDtypeStruct((N, C), jnp.float32),
          jax.ShapeDtypeStruct((N, ST), jnp.float32),
      ),
      mesh=mesh,
      compiler_params=pltpu.CompilerParams(use_tc_tiling_on_sc=False,
                                           needs_layout_passes=False),
      scratch_types=(
          [pltpu.VMEM((TL, BN * K), jnp.int32)]
          + [pltpu.VMEM((BN * K, C // 2), jnp.int32) for _ in range(NB)]
          + [pltpu.VMEM((BN * K, ST), jnp.float32) for _ in range(NB)]
          + [pltpu.VMEM((BN, C), jnp.float32) for _ in range(NB)]
          + [pltpu.VMEM((BN, ST), jnp.float32) for _ in range(NB)]
          + [pltpu.SemaphoreType.DMA for _ in range(2 * NB)]
      ),
  )(knn2d, feati, stat16)


def _tc_dense_body(featb_ref, aggp_ref, ssum_ref, coords_ref,
                   w1_ref, w2_ref, w3_ref, b_ref, out_ref):
  inv_k = 1.0 / K
  s = ssum_ref[...] * inv_k
  m1 = s[:, 0:3]
  m2 = s[:, 3:6]
  rm = m1 - coords_ref[...]
  rs = jnp.sqrt(jnp.maximum(m2 - m1 * m1, 0.0))
  rel = jnp.concatenate([rm, rs], axis=-1)
  aggb = aggp_ref[...].astype(jnp.bfloat16)
  acc = jnp.dot(featb_ref[...], w1_ref[...],
                preferred_element_type=jnp.float32)
  acc += jnp.dot(aggb, w2_ref[...], preferred_element_type=jnp.float32)
  acc += jnp.dot(rel, w3_ref[...], preferred_element_type=jnp.float32)
  acc += b_ref[...]
  out_ref[...] = acc * jax.nn.sigmoid(acc)


def _tc_dense(featb, aggp, ssum, coords, w1b, w2b, w3, b2):
  R = 2000
  grid = (N // R,)
  return pl.pallas_call(
      _tc_dense_body,
      grid=grid,
      in_specs=[
          pl.BlockSpec((R, C), lambda i: (i, 0)),
          pl.BlockSpec((R, C), lambda i: (i, 0)),
          pl.BlockSpec((R, ST), lambda i: (i, 0)),
          pl.BlockSpec((R, 3), lambda i: (i, 0)),
          pl.BlockSpec((C, HIDDEN), lambda i: (0, 0)),
          pl.BlockSpec((C, HIDDEN), lambda i: (0, 0)),
          pl.BlockSpec((6, HIDDEN), lambda i: (0, 0)),
          pl.BlockSpec((1, HIDDEN), lambda i: (0, 0)),
      ],
      out_specs=pl.BlockSpec((R, HIDDEN), lambda i: (i, 0)),
      out_shape=jax.ShapeDtypeStruct((N, HIDDEN), jnp.float32),
  )(featb, aggp, ssum, coords, w1b, w2b, w3, b2)


def kernel(feat, coords, knn_idx, W, b):
  knn2d = knn_idx.astype(jnp.int32).reshape(NBLK, BN * K)
  featb = feat.astype(jnp.bfloat16)
  feati = lax.bitcast_convert_type(
      featb.reshape(N, C // 2, 2), jnp.int32)
  stat16 = jnp.concatenate(
      [coords, coords * coords, jnp.zeros((N, ST - 6), jnp.float32)], axis=1)
  aggp, ssum = _sc_gather(knn2d, feati, stat16)

  w1b = W[0:C].astype(jnp.bfloat16)
  w2b = (W[C:2 * C] * (1.0 / K))[_PERM].astype(jnp.bfloat16)
  w3 = W[2 * C:2 * C + 6]
  b2 = b.reshape(1, HIDDEN)
  return _tc_dense(featb, aggp, ssum, coords, w1b, w2b, w3, b2)


# speedup vs baseline: 1.0500x; 1.0500x over previous
import jax
import jax.numpy as jnp
import numpy as np
from jax import lax
from jax.experimental import pallas as pl
from jax.experimental.pallas import tpu as pltpu
from jax.experimental.pallas import tpu_sc as plsc

N = 10000
C = 128
K = 32
HIDDEN = 128
ST = 16
BN = 4
NBLK = N // BN
NW = 32
TL = 80
NB = 8
LANES = 16
CCH = C // 32

_PERM = np.empty(C, np.int32)
for _c in range(CCH):
  for _q in range(32):
    _PERM[32 * _c + _q] = 32 * _c + (2 * _q if _q < 16 else 2 * (_q - 16) + 1)


def _sc_gather_body(knn_hbm, feat_hbm, stat_hbm, aggp_hbm, ssum_hbm,
                    idx_all, idx_rows, *bufs):
  cid = lax.axis_index("c")
  sid = lax.axis_index("s")
  wid = sid * 2 + cid
  base = jnp.minimum(wid * TL, NBLK - TL)

  rows_b = bufs[0:NB]
  srows_b = bufs[NB:2 * NB]
  agg_b = bufs[2 * NB:3 * NB]
  st_b = bufs[3 * NB:4 * NB]
  semg = bufs[4 * NB:5 * NB]
  semo = bufs[5 * NB:6 * NB]

  pltpu.sync_copy(knn_hbm.at[pl.ds(base, TL)], idx_all)

  def build_idx(t, p):
    for n in range(BN):
      e, o = plsc.unpack(idx_all[t, pl.ds(32 * n, 32)],
                         format=plsc.PackFormat.INTERLEAVED)
      idx_rows[p, pl.ds(32 * n, LANES)] = e
      idx_rows[p, pl.ds(32 * n + LANES, LANES)] = o

  def gathers(t, p):
    return (
        pltpu.make_async_copy(feat_hbm.at[idx_rows.at[p]], rows_b[p], semg[p]),
        pltpu.make_async_copy(stat_hbm.at[idx_rows.at[p]], srows_b[p], semg[p]),
    )

  def out_copies(t, p):
    return (
        pltpu.make_async_copy(
            agg_b[p], aggp_hbm.at[pl.ds((base + t) * BN, BN)], semo[p]),
        pltpu.make_async_copy(
            st_b[p], ssum_hbm.at[pl.ds((base + t) * BN, BN)], semo[p]),
    )

  for tp in range(NB - 1):
    build_idx(tp, tp)
    for cp in gathers(tp, tp):
      cp.start()

  def outer(g, _):
    for phase in range(NB):
      t = g * NB + phase
      tn = t + (NB - 1)

      @pl.when(tn < TL)
      def _():
        build_idx(tn, (phase + NB - 1) % NB)
        for cp in gathers(tn, (phase + NB - 1) % NB):
          cp.start()

      @pl.when(t >= NB)
      def _():
        for cp in out_copies(t - NB, phase):
          cp.wait()

      for cp in gathers(t, phase):
        cp.wait()

      rows_v = rows_b[phase]
      srows_v = srows_b[phase]
      for n in range(BN):
        def red(j, carry):
          r = n * K + j
          new = []
          for c in range(CCH):
            e, o = plsc.unpack(rows_v[r, pl.ds(32 * c, 32)],
                               format=plsc.PackFormat.INTERLEAVED)
            new.append(carry[2 * c] + e)
            new.append(carry[2 * c + 1] + o)
          new.append(carry[2 * CCH] + srows_v[r, :])
          return tuple(new)

        zero = jnp.zeros((LANES,), jnp.float32)
        init = tuple(zero for _ in range(2 * CCH + 1))
        out = lax.fori_loop(0, K, red, init, unroll=2)
        for c in range(CCH):
          agg_b[phase][n, pl.ds(32 * c, LANES)] = out[2 * c]
          agg_b[phase][n, pl.ds(32 * c + LANES, LANES)] = out[2 * c + 1]
        st_b[phase][n, :] = out[2 * CCH]

      for cp in out_copies(t, phase):
        cp.start()

    return _

  lax.fori_loop(0, TL // NB, outer, None)

  for p in range(NB):
    for cp in out_copies(0, p):
      cp.wait()


def _sc_gather(knn2d, featb, stat16):
  mesh = plsc.VectorSubcoreMesh(core_axis_name="c", subcore_axis_name="s")
  return pl.kernel(
      _sc_gather_body,
      out_type=(
          jax.ShapeDtypeStruct((N, C), jnp.float32),
          jax.ShapeDtypeStruct((N, ST), jnp.float32),
      ),
      mesh=mesh,
      compiler_params=pltpu.CompilerParams(use_tc_tiling_on_sc=False,
                                           needs_layout_passes=False),
      scratch_types=(
          [pltpu.VMEM((TL, BN * K), jnp.int16),
           pltpu.VMEM((NB, BN * K), jnp.int32)]
          + [pltpu.VMEM((BN * K, C), jnp.bfloat16) for _ in range(NB)]
          + [pltpu.VMEM((BN * K, ST), jnp.float32) for _ in range(NB)]
          + [pltpu.VMEM((BN, C), jnp.float32) for _ in range(NB)]
          + [pltpu.VMEM((BN, ST), jnp.float32) for _ in range(NB)]
          + [pltpu.SemaphoreType.DMA for _ in range(2 * NB)]
      ),
  )(knn2d, featb, stat16)


def _tc_dense_body(featb_ref, aggp_ref, ssum_ref, coords_ref,
                   w1_ref, w2_ref, w3_ref, b_ref, out_ref):
  inv_k = 1.0 / K
  s = ssum_ref[...] * inv_k
  m1 = s[:, 0:3]
  m2 = s[:, 3:6]
  rm = m1 - coords_ref[...]
  rs = jnp.sqrt(jnp.maximum(m2 - m1 * m1, 0.0))
  rel = jnp.concatenate([rm, rs], axis=-1)
  aggb = aggp_ref[...].astype(jnp.bfloat16)
  acc = jnp.dot(featb_ref[...], w1_ref[...],
                preferred_element_type=jnp.float32)
  acc += jnp.dot(aggb, w2_ref[...], preferred_element_type=jnp.float32)
  acc += jnp.dot(rel, w3_ref[...], preferred_element_type=jnp.float32)
  acc += b_ref[...]
  out_ref[...] = acc * jax.nn.sigmoid(acc)


def _tc_dense(featb, aggp, ssum, coords, w1b, w2b, w3, b2):
  R = 2000
  grid = (N // R,)
  return pl.pallas_call(
      _tc_dense_body,
      grid=grid,
      in_specs=[
          pl.BlockSpec((R, C), lambda i: (i, 0)),
          pl.BlockSpec((R, C), lambda i: (i, 0)),
          pl.BlockSpec((R, ST), lambda i: (i, 0)),
          pl.BlockSpec((R, 3), lambda i: (i, 0)),
          pl.BlockSpec((C, HIDDEN), lambda i: (0, 0)),
          pl.BlockSpec((C, HIDDEN), lambda i: (0, 0)),
          pl.BlockSpec((6, HIDDEN), lambda i: (0, 0)),
          pl.BlockSpec((1, HIDDEN), lambda i: (0, 0)),
      ],
      out_specs=pl.BlockSpec((R, HIDDEN), lambda i: (i, 0)),
      out_shape=jax.ShapeDtypeStruct((N, HIDDEN), jnp.float32),
  )(featb, aggp, ssum, coords, w1b, w2b, w3, b2)


def kernel(feat, coords, knn_idx, W, b):
  knn2d = knn_idx.astype(jnp.int16).reshape(NBLK, BN * K)
  featb = feat.astype(jnp.bfloat16)
  stat16 = jnp.concatenate(
      [coords, coords * coords, jnp.zeros((N, ST - 6), jnp.float32)], axis=1)
  aggp, ssum = _sc_gather(knn2d, featb, stat16)

  w1b = W[0:C].astype(jnp.bfloat16)
  w2b = (W[C:2 * C] * (1.0 / K))[_PERM].astype(jnp.bfloat16)
  w3 = W[2 * C:2 * C + 6]
  b2 = b.reshape(1, HIDDEN)
  return _tc_dense(featb, aggp, ssum, coords, w1b, w2b, w3, b2)


# speedup vs baseline: 1.1847x; 1.1282x over previous
import jax
import jax.numpy as jnp
import numpy as np
from jax import lax
from jax.experimental import pallas as pl
from jax.experimental.pallas import tpu as pltpu
from jax.experimental.pallas import tpu_sc as plsc

N = 10000
C = 128
K = 32
HIDDEN = 128
ST = 16
BN = 4
NBLK = N // BN
NW = 32
TL = 80
NB = 8
LANES = 16
CCH = C // 32

_PERM = np.empty(C, np.int32)
for _c in range(CCH):
  for _q in range(32):
    _PERM[32 * _c + _q] = 32 * _c + (2 * _q if _q < 16 else 2 * (_q - 16) + 1)


def _sc_gather_body(knn_hbm, feat_hbm, stat_hbm, aggp_hbm, ssum_hbm,
                    idx_all, *bufs):
  cid = lax.axis_index("c")
  sid = lax.axis_index("s")
  wid = sid * 2 + cid
  base = jnp.minimum(wid * TL, NBLK - TL)

  rows_b = bufs[0:NB]
  srows_b = bufs[NB:2 * NB]
  agg_b = bufs[2 * NB:3 * NB]
  st_b = bufs[3 * NB:4 * NB]
  semg = bufs[4 * NB:5 * NB]
  semo = bufs[5 * NB:6 * NB]

  pltpu.sync_copy(knn_hbm.at[pl.ds(base, TL)], idx_all)

  def gathers(t, p):
    return (
        pltpu.make_async_copy(feat_hbm.at[idx_all.at[t]], rows_b[p], semg[p]),
        pltpu.make_async_copy(stat_hbm.at[idx_all.at[t]], srows_b[p], semg[p]),
    )

  def out_copies(t, p):
    return (
        pltpu.make_async_copy(
            agg_b[p], aggp_hbm.at[pl.ds((base + t) * BN, BN)], semo[p]),
        pltpu.make_async_copy(
            st_b[p], ssum_hbm.at[pl.ds((base + t) * BN, BN)], semo[p]),
    )

  for tp in range(NB - 1):
    for cp in gathers(tp, tp):
      cp.start()

  def outer(g, _):
    for phase in range(NB):
      t = g * NB + phase
      tn = t + (NB - 1)

      @pl.when(tn < TL)
      def _():
        for cp in gathers(tn, (phase + NB - 1) % NB):
          cp.start()

      @pl.when(t >= NB)
      def _():
        for cp in out_copies(t - NB, phase):
          cp.wait()

      for cp in gathers(t, phase):
        cp.wait()

      rows_v = rows_b[phase]
      srows_v = srows_b[phase]
      for n in range(BN):
        def red(j, carry):
          r = n * K + j
          new = []
          for c in range(CCH):
            e, o = plsc.unpack(rows_v[r, pl.ds(32 * c, 32)],
                               format=plsc.PackFormat.INTERLEAVED)
            new.append(carry[2 * c] + e)
            new.append(carry[2 * c + 1] + o)
          new.append(carry[2 * CCH] + srows_v[r, :])
          return tuple(new)

        zero = jnp.zeros((LANES,), jnp.float32)
        init = tuple(zero for _ in range(2 * CCH + 1))
        out = lax.fori_loop(0, K, red, init, unroll=2)
        for c in range(CCH):
          agg_b[phase][n, pl.ds(32 * c, LANES)] = out[2 * c]
          agg_b[phase][n, pl.ds(32 * c + LANES, LANES)] = out[2 * c + 1]
        st_b[phase][n, :] = out[2 * CCH]

      for cp in out_copies(t, phase):
        cp.start()

    return _

  lax.fori_loop(0, TL // NB, outer, None)

  for p in range(NB):
    for cp in out_copies(0, p):
      cp.wait()


def _sc_gather(knn2d, featb, stat16):
  mesh = plsc.VectorSubcoreMesh(core_axis_name="c", subcore_axis_name="s")
  return pl.kernel(
      _sc_gather_body,
      out_type=(
          jax.ShapeDtypeStruct((N, C), jnp.float32),
          jax.ShapeDtypeStruct((N, ST), jnp.float32),
      ),
      mesh=mesh,
      compiler_params=pltpu.CompilerParams(use_tc_tiling_on_sc=False,
                                           needs_layout_passes=False),
      scratch_types=(
          [pltpu.VMEM((TL, BN * K), jnp.int32)]
          + [pltpu.VMEM((BN * K, C), jnp.bfloat16) for _ in range(NB)]
          + [pltpu.VMEM((BN * K, ST), jnp.float32) for _ in range(NB)]
          + [pltpu.VMEM((BN, C), jnp.float32) for _ in range(NB)]
          + [pltpu.VMEM((BN, ST), jnp.float32) for _ in range(NB)]
          + [pltpu.SemaphoreType.DMA for _ in range(2 * NB)]
      ),
  )(knn2d, featb, stat16)


def _tc_dense_body(featb_ref, aggp_ref, ssum_ref, coords_ref,
                   w1_ref, w2_ref, w3_ref, b_ref, out_ref):
  inv_k = 1.0 / K
  s = ssum_ref[...] * inv_k
  m1 = s[:, 0:3]
  m2 = s[:, 3:6]
  rm = m1 - coords_ref[...]
  rs = jnp.sqrt(jnp.maximum(m2 - m1 * m1, 0.0))
  rel = jnp.concatenate([rm, rs], axis=-1)
  aggb = aggp_ref[...].astype(jnp.bfloat16)
  acc = jnp.dot(featb_ref[...], w1_ref[...],
                preferred_element_type=jnp.float32)
  acc += jnp.dot(aggb, w2_ref[...], preferred_element_type=jnp.float32)
  acc += jnp.dot(rel, w3_ref[...], preferred_element_type=jnp.float32)
  acc += b_ref[...]
  out_ref[...] = acc * jax.nn.sigmoid(acc)


def _tc_dense(featb, aggp, ssum, coords, w1b, w2b, w3, b2):
  R = 2000
  grid = (N // R,)
  return pl.pallas_call(
      _tc_dense_body,
      grid=grid,
      in_specs=[
          pl.BlockSpec((R, C), lambda i: (i, 0)),
          pl.BlockSpec((R, C), lambda i: (i, 0)),
          pl.BlockSpec((R, ST), lambda i: (i, 0)),
          pl.BlockSpec((R, 3), lambda i: (i, 0)),
          pl.BlockSpec((C, HIDDEN), lambda i: (0, 0)),
          pl.BlockSpec((C, HIDDEN), lambda i: (0, 0)),
          pl.BlockSpec((6, HIDDEN), lambda i: (0, 0)),
          pl.BlockSpec((1, HIDDEN), lambda i: (0, 0)),
      ],
      out_specs=pl.BlockSpec((R, HIDDEN), lambda i: (i, 0)),
      out_shape=jax.ShapeDtypeStruct((N, HIDDEN), jnp.float32),
  )(featb, aggp, ssum, coords, w1b, w2b, w3, b2)


def kernel(feat, coords, knn_idx, W, b):
  knn2d = knn_idx.astype(jnp.int32).reshape(NBLK, BN * K)
  featb = feat.astype(jnp.bfloat16)
  stat16 = jnp.concatenate(
      [coords, coords * coords, jnp.zeros((N, ST - 6), jnp.float32)], axis=1)
  aggp, ssum = _sc_gather(knn2d, featb, stat16)

  w1b = W[0:C].astype(jnp.bfloat16)
  w2b = (W[C:2 * C] * (1.0 / K))[_PERM].astype(jnp.bfloat16)
  w3 = W[2 * C:2 * C + 6]
  b2 = b.reshape(1, HIDDEN)
  return _tc_dense(featb, aggp, ssum, coords, w1b, w2b, w3, b2)
